# baseline (speedup 1.0000x reference)
import functools

import jax
import jax.numpy as jnp
from jax import lax
from jax.experimental import pallas as pl
from jax.experimental.pallas import tpu as pltpu
from jax.experimental.pallas import tpu_sc as plsc

N = 10000
E = 160000
D = 256
LANES = 16
NUM_TILES = 32
EPT = E // NUM_TILES
VECS = (EPT + LANES - 1) // LANES
CAP = 5120
WCH = 512
HCH = 64
GCAP = NUM_TILES * HCH
ATHR = -104.0

_MESH = plsc.VectorSubcoreMesh(core_axis_name="c", subcore_axis_name="s")

_SC_PARAMS = pltpu.CompilerParams(
    needs_layout_passes=False, use_tc_tiling_on_sc=False)


def _edge_body(d0_hbm, d1_hbm, d2_hbm, src_hbm, dst_hbm, par_hbm,
               counts_hbm, hsrc_hbm, hdst_hbm, hval_hbm,
               csrc_hbm, cdst_hbm, cval_hbm,
               d0_v, d1_v, d2_v, src_v, dst_v, par_v,
               csrc_v, cdst_v, cval_v, cnt_v, sem):
    c = lax.axis_index("c")
    s = lax.axis_index("s")
    w = c * 16 + s
    ebase = w * EPT
    cps = [
        pltpu.async_copy(d0_hbm, d0_v, sem),
        pltpu.async_copy(d1_hbm, d1_v, sem),
        pltpu.async_copy(d2_hbm, d2_v, sem),
        pltpu.async_copy(src_hbm.at[pl.ds(ebase, EPT)],
                         src_v.at[pl.ds(0, EPT)], sem),
        pltpu.async_copy(dst_hbm.at[pl.ds(ebase, EPT)],
                         dst_v.at[pl.ds(0, EPT)], sem),
        pltpu.async_copy(par_hbm, par_v, sem),
    ]
    for cp in cps:
        cp.wait()
    mu0 = par_v[0, :]
    mu1 = par_v[1, :]
    mu2 = par_v[2, :]
    msig = par_v[3, :]
    lane = lax.iota(jnp.int32, LANES)

    def body(i, off):
        sl = pl.ds(i * LANES, LANES)
        valid = (i * LANES + lane) < EPT
        sv = jnp.where(valid, src_v[sl], 0)
        dv = jnp.where(valid, dst_v[sl], 0)
        a0 = plsc.load_gather(d0_v, [sv])
        a1 = plsc.load_gather(d1_v, [sv])
        a2 = plsc.load_gather(d2_v, [sv])
        b0 = plsc.load_gather(d0_v, [dv])
        b1 = plsc.load_gather(d1_v, [dv])
        b2 = plsc.load_gather(d2_v, [dv])
        t0 = (a0 - b0) - mu0
        t1 = (a1 - b1) - mu1
        t2 = (a2 - b2) - mu2
        arg = (t0 * t0 + t1 * t1 + t2 * t2) * msig
        mask = (arg >= ATHR) & valid
        mi = mask.astype(jnp.int32)
        nsurv = jnp.sum(mi, axis=0)

        @pl.when(nsurv > 0)
        def _():
            val = jnp.exp(arg)
            pos = plsc.cumsum(mi)
            idx = (off + pos) - 1
            plsc.store_scatter(csrc_v, [idx], sv, mask=mask)
            plsc.store_scatter(cdst_v, [idx], dv, mask=mask)
            plsc.store_scatter(cval_v, [idx], val, mask=mask)

        return off + nsurv

    off = lax.fori_loop(0, VECS, body, jnp.int32(0), unroll=2)
    cnt_v[...] = jnp.where(lane == 0, off, 0)
    pltpu.sync_copy(cnt_v, counts_hbm.at[w])
    pltpu.sync_copy(csrc_v.at[pl.ds(0, HCH)], hsrc_hbm.at[w])
    pltpu.sync_copy(cdst_v.at[pl.ds(0, HCH)], hdst_hbm.at[w])
    pltpu.sync_copy(cval_v.at[pl.ds(0, HCH)], hval_hbm.at[w])

    def wb(ci, _):
        slw = pl.ds(ci * WCH, WCH)
        pltpu.sync_copy(csrc_v.at[slw], csrc_hbm.at[w, slw])
        pltpu.sync_copy(cdst_v.at[slw], cdst_hbm.at[w, slw])
        pltpu.sync_copy(cval_v.at[slw], cval_hbm.at[w, slw])
        return 0

    lax.fori_loop(0, (jnp.maximum(off - HCH, 0) + WCH - 1) // WCH, wb, 0)


def _edge_weights(d0, d1, d2, src, dst, params):
    f = pl.kernel(
        _edge_body,
        out_type=(
            jax.ShapeDtypeStruct((NUM_TILES, LANES), jnp.int32),
            jax.ShapeDtypeStruct((NUM_TILES, HCH), jnp.int32),
            jax.ShapeDtypeStruct((NUM_TILES, HCH), jnp.int32),
            jax.ShapeDtypeStruct((NUM_TILES, HCH), jnp.float32),
            jax.ShapeDtypeStruct((NUM_TILES, CAP), jnp.int32),
            jax.ShapeDtypeStruct((NUM_TILES, CAP), jnp.int32),
            jax.ShapeDtypeStruct((NUM_TILES, CAP), jnp.float32),
        ),
        mesh=_MESH,
        compiler_params=_SC_PARAMS,
        scratch_types=[
            pltpu.VMEM((N,), jnp.float32),
            pltpu.VMEM((N,), jnp.float32),
            pltpu.VMEM((N,), jnp.float32),
            pltpu.VMEM((VECS * LANES,), jnp.int32),
            pltpu.VMEM((VECS * LANES,), jnp.int32),
            pltpu.VMEM((4, LANES), jnp.float32),
            pltpu.VMEM((CAP,), jnp.int32),
            pltpu.VMEM((CAP,), jnp.int32),
            pltpu.VMEM((CAP,), jnp.float32),
            pltpu.VMEM((LANES,), jnp.int32),
            pltpu.SemaphoreType.DMA,
        ],
    )
    return f(d0, d1, d2, src, dst, params)


def _tc_round_body(first, cnts_ref, hsrc_ref, hdst_ref, hval_ref, roff_ref,
                   x_ref, w_ref, bias_ref, prev_ref, out_ref, flag_ref):
    roff = roff_ref[0]

    def ovf(t, f):
        return f | (cnts_ref[t, 0] - roff > HCH).astype(jnp.int32)

    flag_ref[0] = lax.fori_loop(0, NUM_TILES, ovf, jnp.int32(0))

    if first:
        out_ref[...] = jnp.broadcast_to(bias_ref[...][None, :], (N, D))
    else:
        out_ref[...] = prev_ref[...]

    for t in range(NUM_TILES):
        m = jnp.clip(cnts_ref[t, 0] - roff, 0, HCH)

        def e_body(k, _, t=t):
            dst_s = hdst_ref[t, k]
            src_s = hsrc_ref[t, k]
            val_s = hval_ref[t, k]
            xrow = x_ref[pl.ds(dst_s, 1), :]
            srow = lax.dot_general(
                xrow, w_ref[...], (((1,), (0,)), ((), ())),
                preferred_element_type=jnp.float32,
                precision=lax.Precision.HIGHEST,
            )
            out_ref[pl.ds(src_s, 1), :] += val_s * srow
            return 0

        lax.fori_loop(0, m, e_body, 0)


def _tc_round(first, counts, hsrc, hdst, hval, roff, x, weight, bias, prev):
    body = functools.partial(_tc_round_body, first)
    return pl.pallas_call(
        body,
        in_specs=[
            pl.BlockSpec(memory_space=pltpu.SMEM),
            pl.BlockSpec(memory_space=pltpu.SMEM),
            pl.BlockSpec(memory_space=pltpu.SMEM),
            pl.BlockSpec(memory_space=pltpu.SMEM),
            pl.BlockSpec(memory_space=pltpu.SMEM),
            pl.BlockSpec(memory_space=pltpu.VMEM),
            pl.BlockSpec(memory_space=pltpu.VMEM),
            pl.BlockSpec(memory_space=pltpu.VMEM),
            pl.BlockSpec(memory_space=pltpu.VMEM),
        ],
        out_specs=[
            pl.BlockSpec(memory_space=pltpu.VMEM),
            pl.BlockSpec(memory_space=pltpu.SMEM),
        ],
        out_shape=[
            jax.ShapeDtypeStruct((N, D), jnp.float32),
            jax.ShapeDtypeStruct((1,), jnp.int32),
        ],

    )(counts, hsrc, hdst, hval, roff, x, weight, bias, prev)


def kernel(x, edge_index, weight, bias, mu, sig):
    x = x.astype(jnp.float32)
    ei = edge_index.astype(jnp.int32)
    src = ei[0]
    dst = ei[1]
    d0 = x[:, 0]
    d1 = x[:, 1]
    d2 = x[:, 2]
    params = jnp.broadcast_to(
        jnp.concatenate([mu.astype(jnp.float32),
                         -0.5 * sig.astype(jnp.float32)])[:, None],
        (4, LANES))
    weight = weight.astype(jnp.float32)
    bias = bias.astype(jnp.float32)
    counts, hsrc, hdst, hval, csrc, cdst, cval = _edge_weights(
        d0, d1, d2, src, dst, params)

    roff0 = jnp.zeros((1,), jnp.int32)
    out, flag = _tc_round(True, counts, hsrc, hdst, hval, roff0,
                          x, weight, bias, jnp.zeros((1, 1), jnp.float32))

    def w_cond(state):
        _, _, flag = state
        return flag[0] > 0

    def w_body(state):
        r, prev, _ = state
        ro = jnp.full((1,), r * HCH, jnp.int32)
        hs = lax.dynamic_slice(csrc, (0, r * HCH), (NUM_TILES, HCH))
        hd = lax.dynamic_slice(cdst, (0, r * HCH), (NUM_TILES, HCH))
        hv = lax.dynamic_slice(cval, (0, r * HCH), (NUM_TILES, HCH))
        nxt, fl = _tc_round(False, counts, hs, hd, hv, ro,
                            x, weight, bias, prev)
        return r + 1, nxt, fl

    _, out, _ = lax.while_loop(w_cond, w_body, (jnp.int32(1), out, flag))
    return out


# speedup vs baseline: 1.0338x; 1.0338x over previous
import functools

import jax
import jax.numpy as jnp
from jax import lax
from jax.experimental import pallas as pl
from jax.experimental.pallas import tpu as pltpu
from jax.experimental.pallas import tpu_sc as plsc

N = 10000
E = 160000
D = 256
LANES = 16
NUM_TILES = 32
EPT = E // NUM_TILES
VECS = (EPT + LANES - 1) // LANES
CAP = 5120
WCH = 512
HCH = 64
GCAP = NUM_TILES * HCH
ATHR = -104.0

_MESH = plsc.VectorSubcoreMesh(core_axis_name="c", subcore_axis_name="s")

_SC_PARAMS = pltpu.CompilerParams(
    needs_layout_passes=False, use_tc_tiling_on_sc=False)


def _edge_body(d0_hbm, d1_hbm, d2_hbm, src_hbm, dst_hbm, par_hbm,
               counts_hbm, hsrc_hbm, hdst_hbm, hval_hbm,
               csrc_hbm, cdst_hbm, cval_hbm,
               d0_v, d1_v, d2_v, src_v, dst_v, par_v,
               csrc_v, cdst_v, cval_v, cnt_v, sem):
    c = lax.axis_index("c")
    s = lax.axis_index("s")
    w = c * 16 + s
    ebase = w * EPT
    cps = [
        pltpu.async_copy(d0_hbm, d0_v, sem),
        pltpu.async_copy(d1_hbm, d1_v, sem),
        pltpu.async_copy(d2_hbm, d2_v, sem),
        pltpu.async_copy(src_hbm.at[pl.ds(ebase, EPT)],
                         src_v.at[pl.ds(0, EPT)], sem),
        pltpu.async_copy(dst_hbm.at[pl.ds(ebase, EPT)],
                         dst_v.at[pl.ds(0, EPT)], sem),
        pltpu.async_copy(par_hbm, par_v, sem),
    ]
    for cp in cps:
        cp.wait()
    mu0 = par_v[0, :]
    mu1 = par_v[1, :]
    mu2 = par_v[2, :]
    msig = par_v[3, :]
    lane = lax.iota(jnp.int32, LANES)

    def process(sv, dv, valid, offv):
        a0 = plsc.load_gather(d0_v, [sv])
        a1 = plsc.load_gather(d1_v, [sv])
        a2 = plsc.load_gather(d2_v, [sv])
        b0 = plsc.load_gather(d0_v, [dv])
        b1 = plsc.load_gather(d1_v, [dv])
        b2 = plsc.load_gather(d2_v, [dv])
        t0 = (a0 - b0) - mu0
        t1 = (a1 - b1) - mu1
        t2 = (a2 - b2) - mu2
        arg = (t0 * t0 + t1 * t1 + t2 * t2) * msig
        mask = (arg >= ATHR) & valid
        cntv = plsc.all_reduce_population_count(mask)

        @pl.when(cntv[0] > 0)
        def _():
            mi = mask.astype(jnp.int32)
            val = jnp.exp(arg)
            pos = plsc.cumsum(mi)
            idx = (offv + pos) - 1
            plsc.store_scatter(csrc_v, [idx], sv, mask=mask)
            plsc.store_scatter(cdst_v, [idx], dv, mask=mask)
            plsc.store_scatter(cval_v, [idx], val, mask=mask)

        return offv + cntv

    tvec = jnp.ones((LANES,), jnp.bool_)

    def body(i, offv):
        sl = pl.ds(i * LANES, LANES)
        return process(src_v[sl], dst_v[sl], tvec, offv)

    offv = lax.fori_loop(0, VECS - 1, body, jnp.zeros((LANES,), jnp.int32),
                         unroll=2)
    sl = pl.ds((VECS - 1) * LANES, LANES)
    offv = process(jnp.where(lane < EPT % LANES, src_v[sl], 0),
                   jnp.where(lane < EPT % LANES, dst_v[sl], 0),
                   lane < EPT % LANES, offv)
    off = offv[0]
    cnt_v[...] = jnp.where(lane == 0, offv, 0)
    pltpu.sync_copy(cnt_v, counts_hbm.at[w])
    pltpu.sync_copy(csrc_v.at[pl.ds(0, HCH)], hsrc_hbm.at[w])
    pltpu.sync_copy(cdst_v.at[pl.ds(0, HCH)], hdst_hbm.at[w])
    pltpu.sync_copy(cval_v.at[pl.ds(0, HCH)], hval_hbm.at[w])

    def wb(ci, _):
        slw = pl.ds(ci * WCH, WCH)
        pltpu.sync_copy(csrc_v.at[slw], csrc_hbm.at[w, slw])
        pltpu.sync_copy(cdst_v.at[slw], cdst_hbm.at[w, slw])
        pltpu.sync_copy(cval_v.at[slw], cval_hbm.at[w, slw])
        return 0

    lax.fori_loop(0, (jnp.maximum(off - HCH, 0) + WCH - 1) // WCH, wb, 0)


def _edge_weights(d0, d1, d2, src, dst, params):
    f = pl.kernel(
        _edge_body,
        out_type=(
            jax.ShapeDtypeStruct((NUM_TILES, LANES), jnp.int32),
            jax.ShapeDtypeStruct((NUM_TILES, HCH), jnp.int32),
            jax.ShapeDtypeStruct((NUM_TILES, HCH), jnp.int32),
            jax.ShapeDtypeStruct((NUM_TILES, HCH), jnp.float32),
            jax.ShapeDtypeStruct((NUM_TILES, CAP), jnp.int32),
            jax.ShapeDtypeStruct((NUM_TILES, CAP), jnp.int32),
            jax.ShapeDtypeStruct((NUM_TILES, CAP), jnp.float32),
        ),
        mesh=_MESH,
        compiler_params=_SC_PARAMS,
        scratch_types=[
            pltpu.VMEM((N,), jnp.float32),
            pltpu.VMEM((N,), jnp.float32),
            pltpu.VMEM((N,), jnp.float32),
            pltpu.VMEM((VECS * LANES,), jnp.int32),
            pltpu.VMEM((VECS * LANES,), jnp.int32),
            pltpu.VMEM((4, LANES), jnp.float32),
            pltpu.VMEM((CAP,), jnp.int32),
            pltpu.VMEM((CAP,), jnp.int32),
            pltpu.VMEM((CAP,), jnp.float32),
            pltpu.VMEM((LANES,), jnp.int32),
            pltpu.SemaphoreType.DMA,
        ],
    )
    return f(d0, d1, d2, src, dst, params)


def _tc_round_body(first, cnts_ref, hsrc_ref, hdst_ref, hval_ref, roff_ref,
                   x_ref, w_ref, bias_ref, prev_ref, out_ref, flag_ref,
                   xg_ref, gsrc_ref, gval_ref, sem):
    roff = roff_ref[0]

    def seg(t, n):
        m = jnp.clip(cnts_ref[t, 0] - roff, 0, HCH)

        def e_body(k, n, t=t):
            dst_s = hdst_ref[t, k]
            pltpu.make_async_copy(
                x_ref.at[pl.ds(dst_s, 1)], xg_ref.at[pl.ds(n, 1)], sem
            ).start()
            gsrc_ref[n] = hsrc_ref[t, k]
            gval_ref[n] = hval_ref[t, k]
            return n + 1

        return lax.fori_loop(0, m, e_body, n)

    n = lax.fori_loop(0, NUM_TILES, seg, jnp.int32(0))

    def ovf(t, f):
        return f | (cnts_ref[t, 0] - roff > HCH).astype(jnp.int32)

    flag_ref[0] = lax.fori_loop(0, NUM_TILES, ovf, jnp.int32(0))

    if first:
        out_ref[...] = jnp.broadcast_to(bias_ref[...][None, :], (N, D))
    else:
        out_ref[...] = prev_ref[...]

    def drain(i, _):
        pltpu.make_async_copy(
            x_ref.at[pl.ds(0, 1)], xg_ref.at[pl.ds(i, 1)], sem
        ).wait()
        return 0

    lax.fori_loop(0, n, drain, 0)

    def acc(i, _):
        xrow = xg_ref[pl.ds(i, 1), :]
        srow = lax.dot_general(
            xrow, w_ref[...], (((1,), (0,)), ((), ())),
            preferred_element_type=jnp.float32,
            precision=lax.Precision.HIGHEST,
        )
        src_s = gsrc_ref[i]
        out_ref[pl.ds(src_s, 1), :] += gval_ref[i] * srow
        return 0

    lax.fori_loop(0, n, acc, 0)


def _tc_round(first, counts, hsrc, hdst, hval, roff, x, weight, bias, prev):
    body = functools.partial(_tc_round_body, first)
    return pl.pallas_call(
        body,
        in_specs=[
            pl.BlockSpec(memory_space=pltpu.SMEM),
            pl.BlockSpec(memory_space=pltpu.SMEM),
            pl.BlockSpec(memory_space=pltpu.SMEM),
            pl.BlockSpec(memory_space=pltpu.SMEM),
            pl.BlockSpec(memory_space=pltpu.SMEM),
            pl.BlockSpec(memory_space=pl.ANY),
            pl.BlockSpec(memory_space=pltpu.VMEM),
            pl.BlockSpec(memory_space=pltpu.VMEM),
            pl.BlockSpec(memory_space=pltpu.VMEM),
        ],
        out_specs=[
            pl.BlockSpec(memory_space=pltpu.VMEM),
            pl.BlockSpec(memory_space=pltpu.SMEM),
        ],
        out_shape=[
            jax.ShapeDtypeStruct((N, D), jnp.float32),
            jax.ShapeDtypeStruct((1,), jnp.int32),
        ],
        scratch_shapes=[
            pltpu.VMEM((GCAP, D), jnp.float32),
            pltpu.SMEM((GCAP,), jnp.int32),
            pltpu.SMEM((GCAP,), jnp.float32),
            pltpu.SemaphoreType.DMA,
        ],

    )(counts, hsrc, hdst, hval, roff, x, weight, bias, prev)


def kernel(x, edge_index, weight, bias, mu, sig):
    x = x.astype(jnp.float32)
    ei = edge_index.astype(jnp.int32)
    src = ei[0]
    dst = ei[1]
    d0 = x[:, 0]
    d1 = x[:, 1]
    d2 = x[:, 2]
    params = jnp.broadcast_to(
        jnp.concatenate([mu.astype(jnp.float32),
                         -0.5 * sig.astype(jnp.float32)])[:, None],
        (4, LANES))
    weight = weight.astype(jnp.float32)
    bias = bias.astype(jnp.float32)
    counts, hsrc, hdst, hval, csrc, cdst, cval = _edge_weights(
        d0, d1, d2, src, dst, params)

    roff0 = jnp.zeros((1,), jnp.int32)
    out, flag = _tc_round(True, counts, hsrc, hdst, hval, roff0,
                          x, weight, bias, jnp.zeros((1, 1), jnp.float32))

    def w_cond(state):
        _, _, flag = state
        return flag[0] > 0

    def w_body(state):
        r, prev, _ = state
        ro = jnp.full((1,), r * HCH, jnp.int32)
        hs = lax.dynamic_slice(csrc, (0, r * HCH), (NUM_TILES, HCH))
        hd = lax.dynamic_slice(cdst, (0, r * HCH), (NUM_TILES, HCH))
        hv = lax.dynamic_slice(cval, (0, r * HCH), (NUM_TILES, HCH))
        nxt, fl = _tc_round(False, counts, hs, hd, hv, ro,
                            x, weight, bias, prev)
        return r + 1, nxt, fl

    _, out, _ = lax.while_loop(w_cond, w_body, (jnp.int32(1), out, flag))
    return out
